# baseline (speedup 1.0000x reference)
import functools

import jax
import jax.numpy as jnp
from jax import lax
from jax.experimental import pallas as pl
from jax.experimental.pallas import tpu as pltpu, tpu_sc as plsc

NUM_CLASSES = 1000
CAP = 32
DIM = 1024
BATCH = 4096

SLOTS = NUM_CLASSES * CAP
NUM_TILES = 32
SPT = SLOTS // NUM_TILES
K = 48
CH_SZ = [K] * 20 + [40]
CH_OFF = [sum(CH_SZ[:i]) for i in range(len(CH_SZ))]
NCH = len(CH_SZ)
L = 16
NCHUNKS = BATCH // L
NIDX = 63


def _sc_update(mem_flat, x, classes):
    mesh = plsc.VectorSubcoreMesh(core_axis_name="c", subcore_axis_name="s")

    @functools.partial(
        pl.kernel,
        out_type=jax.ShapeDtypeStruct((SLOTS, DIM), jnp.float32),
        mesh=mesh,
        compiler_params=pltpu.CompilerParams(needs_layout_passes=False),
        scratch_types=[
            pltpu.VMEM((BATCH,), jnp.int32),
            pltpu.VMEM((48,), jnp.int32),
            pltpu.VMEM((NIDX * L,), jnp.int32),
            pltpu.VMEM((64, L), jnp.int32),
            pltpu.VMEM((64, L), jnp.int32),
            pltpu.VMEM((K, DIM), jnp.float32),
            pltpu.VMEM((K, DIM), jnp.float32),
            pltpu.SemaphoreType.DMA,
            pltpu.SemaphoreType.DMA,
            pltpu.SemaphoreType.DMA,
            pltpu.SemaphoreType.DMA,
            pltpu.SemaphoreType.DMA,
            pltpu.SemaphoreType.DMA,
            pltpu.SemaphoreType.DMA,
            pltpu.SemaphoreType.DMA,
            pltpu.SemaphoreType.DMA,
            pltpu.SemaphoreType.DMA,
            pltpu.SemaphoreType.DMA,
            pltpu.SemaphoreType.DMA,
        ],
    )
    def k(mem_hbm, x_hbm, cls_hbm, out_hbm,
          cls_v, counts_v, idx_v, src_list, dst_list, rows_a, rows_b,
          gs0, gs1, ws0, ws1, ps0, ps1, ps2, ps3, ss0, ss1, ss2, ss3):
        g = lax.axis_index("c") * 16 + lax.axis_index("s")
        s0 = g * SPT
        c0 = s0 // CAP
        iota = lax.iota(jnp.int32, L)
        zeros = jnp.zeros((L,), jnp.int32)

        pltpu.sync_copy(cls_hbm, cls_v)
        for i in range(3):
            counts_v[pl.ds(i * L, L)] = zeros

        def scan_body(t, n):
            cvec = cls_v[pl.ds(t * L, L)]
            lc = cvec - c0
            own = (lc >= 0) & (lc < 33)

            def active(n):
                lcs = jnp.where(own, lc, 33)
                prior = plsc.load_gather(counts_v, [lcs])
                run, last_m = plsc.scan_count(cvec)
                runi = run.astype(jnp.int32)
                rank = prior + runi - 1
                plsc.store_scatter(counts_v, [lcs], prior + runi,
                                   mask=last_m)
                dstv = cvec * CAP + rank
                valid = own & (rank < CAP) & (dstv >= s0) & (dstv < s0 + SPT)
                pfx = plsc.cumsum(valid.astype(jnp.int32))
                posv = n + pfx - 1
                plsc.store_scatter(src_list, [posv // L, posv % L],
                                   t * L + iota, mask=valid)
                plsc.store_scatter(dst_list, [posv // L, posv % L],
                                   dstv, mask=valid)
                return n + pfx[15]

            return lax.cond(jnp.any(own), active, lambda n: n, n)

        n = lax.fori_loop(0, NCHUNKS, scan_body, 0)

        def idx_body(i, carry):
            slots = s0 + i * L + iota
            cls_of = slots // CAP
            j = slots - cls_of * CAP
            cnt = plsc.load_gather(counts_v, [cls_of - c0])
            idx_v[pl.ds(i * L, L)] = (
                cls_of * CAP + jnp.clip(j - cnt, 0, CAP - 1))
            return carry

        lax.fori_loop(0, NIDX, idx_body, 0)

        rows = [rows_a, rows_b]
        gsem = [gs0, gs1]
        wsem = [ws0, ws1]
        gh = [None, None]
        wh = [None, None]

        def start_gather(ch):
            b = ch % 2
            sz = CH_SZ[ch]
            gh[b] = pltpu.async_copy(
                mem_hbm.at[idx_v.at[pl.ds(CH_OFF[ch], sz)]],
                rows[b].at[pl.ds(0, sz)], gsem[b])

        start_gather(0)
        for ch in range(NCH):
            b = ch % 2
            sz = CH_SZ[ch]
            if ch + 1 < NCH:
                if wh[1 - b] is not None:
                    wh[1 - b].wait()
                start_gather(ch + 1)
            gh[b].wait()
            wh[b] = pltpu.async_copy(rows[b].at[pl.ds(0, sz)],
                                     out_hbm.at[pl.ds(s0 + CH_OFF[ch], sz)],
                                     wsem[b])
        for b in range(2):
            if wh[b] is not None:
                wh[b].wait()

        nsub_pad = ((n + 63) // 64) * 4

        @pl.when(n > 0)
        def _():
            def fill_row(r, carry):
                pos = jnp.minimum(r * L + iota, n - 1)
                vs = plsc.load_gather(src_list, [pos // L, pos % L])
                vd = plsc.load_gather(dst_list, [pos // L, pos % L])
                rv = jnp.full((L,), r, jnp.int32)
                plsc.store_scatter(src_list, [rv, iota], vs)
                plsc.store_scatter(dst_list, [rv, iota], vd)
                return carry

            lax.fori_loop((n - 1) // L, nsub_pad, fill_row, 0)

        bufs = [rows_a.at[pl.ds(q * L, L)] for q in range(3)]
        bufs.append(rows_b.at[pl.ds(0, L)])
        psems = [ps0, ps1, ps2, ps3]
        ssems = [ss0, ss1, ss2, ss3]

        def quad_body(u, carry):
            t = u * 4
            ghs = [pltpu.async_copy(x_hbm.at[src_list.at[t + q]], bufs[q],
                                    psems[q]) for q in range(4)]
            shs = []
            for q in range(4):
                ghs[q].wait()
                shs.append(pltpu.async_copy(
                    bufs[q], out_hbm.at[dst_list.at[t + q]], ssems[q]))
            for q in range(4):
                shs[q].wait()
            return carry

        lax.fori_loop(0, nsub_pad // 4, quad_body, 0)

    return k(mem_flat, x, classes)


def kernel(x, classes, memory):
    out = _sc_update(memory.reshape(SLOTS, DIM), x, classes)
    return out.reshape(NUM_CLASSES, CAP, DIM)


# speedup vs baseline: 1.1101x; 1.1101x over previous
import functools

import jax
import jax.numpy as jnp
from jax import lax
from jax.experimental import pallas as pl
from jax.experimental.pallas import tpu as pltpu, tpu_sc as plsc

NUM_CLASSES = 1000
CAP = 32
DIM = 1024
BATCH = 4096

SLOTS = NUM_CLASSES * CAP
NUM_TILES = 32
SPT = SLOTS // NUM_TILES
K = 48
CH_SZ = [K] * 20 + [40]
CH_OFF = [sum(CH_SZ[:i]) for i in range(len(CH_SZ))]
NCH = len(CH_SZ)
L = 16
NCHUNKS = BATCH // L
NIDX = 63


def _sc_update(mem_flat, x, classes):
    mesh = plsc.VectorSubcoreMesh(core_axis_name="c", subcore_axis_name="s")

    @functools.partial(
        pl.kernel,
        out_type=jax.ShapeDtypeStruct((SLOTS, DIM), jnp.float32),
        mesh=mesh,
        compiler_params=pltpu.CompilerParams(needs_layout_passes=False),
        scratch_types=[
            pltpu.VMEM((BATCH,), jnp.int32),
            pltpu.VMEM((48,), jnp.int32),
            pltpu.VMEM((NIDX * L,), jnp.int32),
            pltpu.VMEM((NIDX * L,), jnp.int32),
            pltpu.VMEM((64, L), jnp.int32),
            pltpu.VMEM((64, L), jnp.int32),
            pltpu.VMEM((K, DIM), jnp.float32),
            pltpu.VMEM((K, DIM), jnp.float32),
            pltpu.SemaphoreType.DMA,
            pltpu.SemaphoreType.DMA,
            pltpu.SemaphoreType.DMA,
            pltpu.SemaphoreType.DMA,
            pltpu.SemaphoreType.DMA,
            pltpu.SemaphoreType.DMA,
            pltpu.SemaphoreType.DMA,
            pltpu.SemaphoreType.DMA,
        ],
    )
    def k(mem_hbm, x_hbm, cls_hbm, out_hbm,
          cls_v, counts_v, idx_v, xsrc_tab, src_list, dst_list, rows_a, rows_b,
          gs0, gs1, ws0, ws1, ps0, ps1, ss0, ss1):
        g = lax.axis_index("c") * 16 + lax.axis_index("s")
        s0 = g * SPT
        c0 = s0 // CAP
        iota = lax.iota(jnp.int32, L)
        zeros = jnp.zeros((L,), jnp.int32)

        pltpu.sync_copy(cls_hbm, cls_v)
        for i in range(3):
            counts_v[pl.ds(i * L, L)] = zeros

        def scan_body(t, n):
            cvec = cls_v[pl.ds(t * L, L)]
            lc = cvec - c0
            own = (lc >= 0) & (lc < 33)

            def active(n):
                lcs = jnp.where(own, lc, 33)
                prior = plsc.load_gather(counts_v, [lcs])
                run, last_m = plsc.scan_count(cvec)
                runi = run.astype(jnp.int32)
                rank = prior + runi - 1
                plsc.store_scatter(counts_v, [lcs], prior + runi,
                                   mask=last_m)
                dstv = cvec * CAP + rank
                valid = own & (rank < CAP) & (dstv >= s0) & (dstv < s0 + SPT)
                plsc.store_scatter(xsrc_tab, [dstv - s0], t * L + iota,
                                   mask=valid)
                return n

            return lax.cond(jnp.any(own), active, lambda n: n, n)

        lax.fori_loop(0, NCHUNKS, scan_body, 0)

        def idx_body(i, m):
            slots = s0 + i * L + iota
            ok = slots < s0 + SPT
            cls_of = slots // CAP
            j = slots - cls_of * CAP
            cnt = plsc.load_gather(counts_v, [cls_of - c0])
            idx_v[pl.ds(i * L, L)] = (
                cls_of * CAP + jnp.clip(j - cnt, 0, CAP - 1))
            valid = (j < cnt) & ok
            pfx = plsc.cumsum(valid.astype(jnp.int32))
            posv = m + pfx - 1
            srcs = plsc.load_gather(xsrc_tab, [slots - s0])
            plsc.store_scatter(src_list, [posv // L, posv % L], srcs,
                               mask=valid)
            plsc.store_scatter(dst_list, [posv // L, posv % L], slots,
                               mask=valid)
            return m + pfx[15]

        n = lax.fori_loop(0, NIDX, idx_body, 0)

        rows = [rows_a, rows_b]
        gsem = [gs0, gs1]
        wsem = [ws0, ws1]
        gh = [None, None]
        wh = [None, None]

        def start_gather(ch):
            b = ch % 2
            sz = CH_SZ[ch]
            gh[b] = pltpu.async_copy(
                mem_hbm.at[idx_v.at[pl.ds(CH_OFF[ch], sz)]],
                rows[b].at[pl.ds(0, sz)], gsem[b])

        start_gather(0)
        for ch in range(NCH):
            b = ch % 2
            sz = CH_SZ[ch]
            if ch + 1 < NCH:
                if wh[1 - b] is not None:
                    wh[1 - b].wait()
                start_gather(ch + 1)
            gh[b].wait()
            wh[b] = pltpu.async_copy(rows[b].at[pl.ds(0, sz)],
                                     out_hbm.at[pl.ds(s0 + CH_OFF[ch], sz)],
                                     wsem[b])
        for b in range(2):
            if wh[b] is not None:
                wh[b].wait()

        @pl.when(n > 0)
        def _():
            r = (n - 1) // L
            rv = jnp.full((L,), r, jnp.int32)
            pos = jnp.minimum(r * L + iota, n - 1)
            vs = plsc.load_gather(src_list, [pos // L, pos % L])
            vd = plsc.load_gather(dst_list, [pos // L, pos % L])
            plsc.store_scatter(src_list, [rv, iota], vs)
            plsc.store_scatter(dst_list, [rv, iota], vd)

        nsub = (n + (L - 1)) // L
        buf_a = rows_a.at[pl.ds(0, L)]
        buf_b = rows_a.at[pl.ds(L, L)]

        def pair_body(u, carry):
            t = u * 2
            ga = pltpu.async_copy(x_hbm.at[src_list.at[t]], buf_a, ps0)
            has_b = t + 1 < nsub

            @pl.when(has_b)
            def _():
                gb = pltpu.async_copy(x_hbm.at[src_list.at[t + 1]],
                                      buf_b, ps1)
                ga.wait()
                sa = pltpu.async_copy(buf_a, out_hbm.at[dst_list.at[t]], ss0)
                gb.wait()
                sb = pltpu.async_copy(buf_b, out_hbm.at[dst_list.at[t + 1]],
                                      ss1)
                sa.wait()
                sb.wait()

            @pl.when(jnp.logical_not(has_b))
            def _():
                ga.wait()
                pltpu.async_copy(buf_a, out_hbm.at[dst_list.at[t]],
                                 ss0).wait()

            return carry

        lax.fori_loop(0, (nsub + 1) // 2, pair_body, 0)

    return k(mem_flat, x, classes)


def kernel(x, classes, memory):
    out = _sc_update(memory.reshape(SLOTS, DIM), x, classes)
    return out.reshape(NUM_CLASSES, CAP, DIM)
